# baseline (speedup 1.0000x reference)
import functools

import jax
import jax.numpy as jnp
from jax.experimental import pallas as pl
from jax.experimental.pallas import tpu as pltpu

SCALE = 30.0
MARGIN = 0.35
LANES = 128


def _accumulate(chunks, masks, m_scr, s_scr):
    bm = functools.reduce(
        jnp.maximum,
        [c if mk is None else jnp.where(mk, c, -jnp.inf)
         for c, mk in zip(chunks, masks)],
    )
    m_old = m_scr[...]
    m_new = jnp.maximum(m_old, bm)
    acc = s_scr[...] * jnp.where(
        m_old == -jnp.inf, 0.0, jnp.exp(m_old - m_new)
    )
    for c, mk in zip(chunks, masks):
        p = jnp.exp(c - m_new)
        acc += p if mk is None else jnp.where(mk, p, 0.0)
    m_scr[...] = m_new
    s_scr[...] = acc


def _lmcl_body(C, W, B, x_ref, tgt_ref, o_ref, m_scr, s_scr, xt_scr):
    j = pl.program_id(0)
    nj = pl.num_programs(0)
    nch = W // LANES

    @pl.when(j == 0)
    def _init():
        m_scr[...] = jnp.full((B, LANES), -jnp.inf, jnp.float32)
        s_scr[...] = jnp.zeros((B, LANES), jnp.float32)
        xt_scr[...] = jnp.zeros((B, LANES), jnp.float32)

    yb = x_ref[...] * SCALE
    chunks = [yb[:, k * LANES:(k + 1) * LANES] for k in range(nch)]
    lane = jax.lax.broadcasted_iota(jnp.int32, (B, LANES), 1)
    tgt = tgt_ref[...]

    xt_acc = xt_scr[...]
    for k in range(nch):
        eq = lane == tgt - (j * W + k * LANES)
        xt_acc += jnp.where(eq, chunks[k], 0.0)
    xt_scr[...] = xt_acc

    @pl.when(j < nj - 1)
    def _full():
        _accumulate(chunks, [None] * nch, m_scr, s_scr)

    @pl.when(j == nj - 1)
    def _last():
        last_valid = C - (nj - 1) * W
        lchunks, lmasks = [], []
        for k in range(nch):
            base = k * LANES
            if base >= last_valid:
                continue
            lchunks.append(chunks[k])
            lmasks.append(
                None if base + LANES <= last_valid
                else lane < last_valid - base
            )
        _accumulate(lchunks, lmasks, m_scr, s_scr)

        m128 = m_scr[...]
        s128 = s_scr[...]
        m = jnp.max(m128, axis=1, keepdims=True)
        s = jnp.sum(s128 * jnp.exp(m128 - m), axis=1, keepdims=True)
        yt = jnp.sum(xt_scr[...], axis=1, keepdims=True)
        ytm = yt - SCALE * MARGIN
        s_corr = s - jnp.exp(yt - m) + jnp.exp(ytm - m)
        nll = m + jnp.log(s_corr) - ytm
        o_ref[...] = jnp.sum(nll, axis=0, keepdims=True) / B


def kernel(output, target):
    B, C = output.shape
    W = 2048
    nj = pl.cdiv(C, W)
    tgt = target.astype(jnp.int32).reshape(B, 1)

    out = pl.pallas_call(
        functools.partial(_lmcl_body, C, W, B),
        grid=(nj,),
        in_specs=[
            pl.BlockSpec((B, W), lambda j: (0, j)),
            pl.BlockSpec((B, 1), lambda j: (0, 0)),
        ],
        out_specs=pl.BlockSpec((1, 1), lambda j: (0, 0)),
        out_shape=jax.ShapeDtypeStruct((1, 1), jnp.float32),
        scratch_shapes=[
            pltpu.VMEM((B, LANES), jnp.float32),
            pltpu.VMEM((B, LANES), jnp.float32),
            pltpu.VMEM((B, LANES), jnp.float32),
        ],
    )(output, tgt)
    return out[0, 0]
